# baseline (speedup 1.0000x reference)
import functools

import jax
import jax.numpy as jnp
from jax import lax
from jax.experimental import pallas as pl
from jax.experimental.pallas import tpu as pltpu
from jax.experimental.pallas import tpu_sc as plsc

B, C, X, Y = 4, 96, 256, 256
N = 131072
BXY = B * X * Y
NC, NS, L = 2, 16, 16
NW = NC * NS
PTS_PER_W = N // NW
DST_PER_W = BXY // NW
ROWS_PER_W = DST_PER_W // Y
PCHUNK = 8192


def _p_index_body(coords_hbm, p_hbm, cbuf, pout):
    w = lax.axis_index("s") * NC + lax.axis_index("c")
    base = w * PTS_PER_W
    pltpu.sync_copy(coords_hbm.at[pl.ds(base * 3, PTS_PER_W * 3)], cbuf)
    iota = lax.iota(jnp.int32, L)

    @plsc.parallel_loop(0, PTS_PER_W // L, unroll=4)
    def _(j):
        r3 = (j * L + iota) * 3
        bb = plsc.load_gather(cbuf, [r3])
        xx = plsc.load_gather(cbuf, [r3 + 1])
        yy = plsc.load_gather(cbuf, [r3 + 2])
        pout[pl.ds(j * L, L)] = (bb * X + xx) * Y + yy

    pltpu.sync_copy(pout, p_hbm.at[pl.ds(base, PTS_PER_W)])


def _dense_body(p_hbm, feats_hbm, out_hbm, idxbuf, pbuf, rows, outb, fidx,
                maskf, sem):
    w = lax.axis_index("s") * NC + lax.axis_index("c")
    dbase = w * DST_PER_W
    b = w // (X // ROWS_PER_W)
    x0 = (w % (X // ROWS_PER_W)) * ROWS_PER_W
    iota = lax.iota(jnp.int32, L)
    zero16 = jnp.zeros((L,), jnp.int32)

    with jax.named_scope("clear"):
        @plsc.parallel_loop(0, DST_PER_W // L, unroll=8)
        def _(g):
            idxbuf[pl.ds(g * L, L)] = zero16

    with jax.named_scope("scan"):
        for chunk in range(N // PCHUNK):
            pltpu.sync_copy(p_hbm.at[pl.ds(chunk * PCHUNK, PCHUNK)], pbuf)
            cbase = chunk * PCHUNK + 1

            @plsc.parallel_loop(0, PCHUNK // L, unroll=4)
            def _(j):
                v = pbuf[pl.ds(j * L, L)]
                rel = v - dbase
                m = (rel >= 0) & (rel < DST_PER_W)
                relc = jnp.clip(rel, 0, DST_PER_W - 1)
                plsc.store_scatter(idxbuf, [relc], cbase + j * L + iota,
                                   mask=m)

    def row_body(sb, _):
        rbase = sb * Y
        with jax.named_scope("prep"):
            for g in range(Y // L):
                v = idxbuf[pl.ds(rbase + g * L, L)]
                fidx[g // 8, pl.ds((g % 8) * L, L)] = jnp.maximum(v - 1, 0)
                maskf[pl.ds(g * L, L)] = jnp.where(v > 0, 1.0, 0.0)
        with jax.named_scope("gather"):
            cp0 = pltpu.async_copy(feats_hbm.at[fidx.at[0]],
                                   rows.at[pl.ds(0, 128), :], sem)
            cp1 = pltpu.async_copy(feats_hbm.at[fidx.at[1]],
                                   rows.at[pl.ds(128, 128), :], sem)
            cp0.wait()
            cp1.wait()

        with jax.named_scope("transpose"):
            def g_body(g, _):
                mv = maskf[pl.ds(g * L, L)]
                posv = g * L + iota

                @plsc.parallel_loop(0, C, unroll=8)
                def _(c):
                    vals = plsc.load_gather(rows, [posv, zero16 + c])
                    outb[c, pl.ds(g * L, L)] = vals * mv

                return 0

            lax.fori_loop(0, Y // L, g_body, 0)
        with jax.named_scope("writeout"):
            pltpu.sync_copy(outb, out_hbm.at[b, :, x0 + sb, :])
        return 0

    lax.fori_loop(0, ROWS_PER_W, row_body, 0)


@functools.cache
def _build():
    mesh = plsc.VectorSubcoreMesh(core_axis_name="c", subcore_axis_name="s")
    cparams = pltpu.CompilerParams(needs_layout_passes=False,
                                   use_tc_tiling_on_sc=False)
    k1 = pl.kernel(
        _p_index_body,
        out_type=jax.ShapeDtypeStruct((N,), jnp.int32),
        mesh=mesh,
        compiler_params=cparams,
        scratch_types=[
            pltpu.VMEM((PTS_PER_W * 3,), jnp.int32),
            pltpu.VMEM((PTS_PER_W,), jnp.int32),
        ],
    )
    k2 = pl.kernel(
        _dense_body,
        out_type=jax.ShapeDtypeStruct((B, C, X, Y), jnp.float32),
        mesh=mesh,
        compiler_params=cparams,
        scratch_types=[
            pltpu.VMEM((DST_PER_W,), jnp.int32),
            pltpu.VMEM((PCHUNK,), jnp.int32),
            pltpu.VMEM((Y, C), jnp.float32),
            pltpu.VMEM((C, Y), jnp.float32),
            pltpu.VMEM((2, 128), jnp.int32),
            pltpu.VMEM((Y,), jnp.float32),
            pltpu.SemaphoreType.DMA,
        ],
    )
    return k1, k2


def kernel(feats, coords):
    k1, k2 = _build()
    coords_flat = coords.astype(jnp.int32).reshape(-1)
    p = k1(coords_flat)
    return k2(p, feats)


# speedup vs baseline: 1.0007x; 1.0007x over previous
import functools

import jax
import jax.numpy as jnp
from jax import lax
from jax.experimental import pallas as pl
from jax.experimental.pallas import tpu as pltpu
from jax.experimental.pallas import tpu_sc as plsc

B, C, X, Y = 4, 96, 256, 256
N = 131072
BXY = B * X * Y
NC, NS, L = 2, 16, 16
NW = NC * NS
PTS_PER_W = N // NW
DST_PER_W = BXY // NW
ROWS_PER_W = DST_PER_W // Y
PCHUNK = 8192
K = 8
RPD = Y // K
GPD = RPD // L


def _p_index_body(coords_hbm, p_hbm, cbuf, pout):
    w = lax.axis_index("s") * NC + lax.axis_index("c")
    base = w * PTS_PER_W
    pltpu.sync_copy(coords_hbm.at[pl.ds(base * 3, PTS_PER_W * 3)], cbuf)
    iota = lax.iota(jnp.int32, L)

    @plsc.parallel_loop(0, PTS_PER_W // L, unroll=4)
    def _(j):
        r3 = (j * L + iota) * 3
        bb = plsc.load_gather(cbuf, [r3])
        xx = plsc.load_gather(cbuf, [r3 + 1])
        yy = plsc.load_gather(cbuf, [r3 + 2])
        pout[pl.ds(j * L, L)] = (bb * X + xx) * Y + yy

    pltpu.sync_copy(pout, p_hbm.at[pl.ds(base, PTS_PER_W)])


def _dense_body(p_hbm, feats_hbm, out_hbm, idxbuf, pbuf, rows, outb, fidx,
                maskf, sem):
    w = lax.axis_index("s") * NC + lax.axis_index("c")
    dbase = w * DST_PER_W
    b = w // (X // ROWS_PER_W)
    x0 = (w % (X // ROWS_PER_W)) * ROWS_PER_W
    iota = lax.iota(jnp.int32, L)
    zero16 = jnp.zeros((L,), jnp.int32)

    with jax.named_scope("clear"):
        @plsc.parallel_loop(0, DST_PER_W // L, unroll=8)
        def _(g):
            idxbuf[pl.ds(g * L, L)] = zero16

    with jax.named_scope("scan"):
        for chunk in range(N // PCHUNK):
            pltpu.sync_copy(p_hbm.at[pl.ds(chunk * PCHUNK, PCHUNK)], pbuf)
            cbase = chunk * PCHUNK + 1

            @plsc.parallel_loop(0, PCHUNK // L, unroll=4)
            def _(j):
                v = pbuf[pl.ds(j * L, L)]
                rel = v - dbase
                m = (rel >= 0) & (rel < DST_PER_W)
                relc = jnp.clip(rel, 0, DST_PER_W - 1)
                plsc.store_scatter(idxbuf, [relc], cbase + j * L + iota,
                                   mask=m)

    def prep(ring, sb):
        rbase = sb * Y
        for g in range(Y // L):
            v = idxbuf[pl.ds(rbase + g * L, L)]
            fidx[ring, g // GPD, pl.ds((g % GPD) * L, L)] = \
                jnp.maximum(v - 1, 0)
            maskf[ring, pl.ds(g * L, L)] = jnp.where(v > 0, 1.0, 0.0)

    def fire(ring):
        for q in range(K):
            pltpu.async_copy(feats_hbm.at[fidx.at[ring, q]],
                             rows.at[ring, pl.ds(q * RPD, RPD), :], sem)

    def drain(ring):
        for q in range(K):
            pltpu.make_async_copy(feats_hbm.at[fidx.at[ring, q]],
                                  rows.at[ring, pl.ds(q * RPD, RPD), :],
                                  sem).wait()

    def flush(ring, sb):
        with jax.named_scope("gwait"):
            drain(ring)
        with jax.named_scope("transpose"):
            def g_body(g, _):
                mv = maskf[ring, pl.ds(g * L, L)]
                posv = g * L + iota

                @plsc.parallel_loop(0, C, unroll=8)
                def _(c):
                    vals = plsc.load_gather(rows.at[ring], [posv, zero16 + c])
                    outb[c, pl.ds(g * L, L)] = vals * mv

                return 0

            lax.fori_loop(0, Y // L, g_body, 0)
        with jax.named_scope("writeout"):
            pltpu.sync_copy(outb, out_hbm.at[b, :, x0 + sb, :])

    with jax.named_scope("prime"):
        prep(0, 0)
        fire(0)

    def pair_body(t, _):
        sb0 = 2 * t
        with jax.named_scope("prep"):
            prep(1, sb0 + 1)
        fire(1)
        flush(0, sb0)

        @pl.when(t < ROWS_PER_W // 2 - 1)
        def _():
            with jax.named_scope("prep"):
                prep(0, sb0 + 2)
            fire(0)

        flush(1, sb0 + 1)
        return 0

    lax.fori_loop(0, ROWS_PER_W // 2, pair_body, 0)


@functools.cache
def _build():
    mesh = plsc.VectorSubcoreMesh(core_axis_name="c", subcore_axis_name="s")
    cparams = pltpu.CompilerParams(needs_layout_passes=False,
                                   use_tc_tiling_on_sc=False)
    k1 = pl.kernel(
        _p_index_body,
        out_type=jax.ShapeDtypeStruct((N,), jnp.int32),
        mesh=mesh,
        compiler_params=cparams,
        scratch_types=[
            pltpu.VMEM((PTS_PER_W * 3,), jnp.int32),
            pltpu.VMEM((PTS_PER_W,), jnp.int32),
        ],
    )
    k2 = pl.kernel(
        _dense_body,
        out_type=jax.ShapeDtypeStruct((B, C, X, Y), jnp.float32),
        mesh=mesh,
        compiler_params=cparams,
        scratch_types=[
            pltpu.VMEM((DST_PER_W,), jnp.int32),
            pltpu.VMEM((PCHUNK,), jnp.int32),
            pltpu.VMEM((2, Y, C), jnp.float32),
            pltpu.VMEM((C, Y), jnp.float32),
            pltpu.VMEM((2, K, RPD), jnp.int32),
            pltpu.VMEM((2, Y), jnp.float32),
            pltpu.SemaphoreType.DMA,
        ],
    )
    return k1, k2


def kernel(feats, coords):
    k1, k2 = _build()
    coords_flat = coords.astype(jnp.int32).reshape(-1)
    p = k1(coords_flat)
    return k2(p, feats)
